# baseline (speedup 1.0000x reference)
import functools

import jax
import jax.numpy as jnp
from jax import lax
from jax.experimental import pallas as pl
from jax.experimental.pallas import tpu as pltpu


def _body(idx_ref, emb_ref, w_ref, b_ref, x_ref, o_ref, t_ref):
    i = pl.program_id(0)

    @pl.when(i == 0)
    def _():
        t_ref[...] = lax.dot_general(
            emb_ref[...], w_ref[...],
            dimension_numbers=(((1,), (1,)), ((), ())),
            preferred_element_type=jnp.float32,
        ) + b_ref[...]

    ids = idx_ref[...] * 2
    oh = (ids == lax.broadcasted_iota(jnp.int32, (1, 100), 1)).astype(jnp.float32)
    t_rows = lax.dot_general(
        oh, t_ref[...],
        dimension_numbers=(((1,), (0,)), ((), ())),
        preferred_element_type=jnp.float32,
    )
    o_ref[...] = x_ref[...] + t_rows[:, None, :]


@functools.partial(jax.jit, static_argnames=("rows_per_block",))
def _run(x3, dts2d, emb_table, w, b2d, rows_per_block):
    B, P, H = x3.shape
    n_blocks = B // rows_per_block
    grid = (n_blocks,)
    return pl.pallas_call(
        _body,
        grid=grid,
        in_specs=[
            pl.BlockSpec((rows_per_block, 1), lambda i: (i, 0)),
            pl.BlockSpec((100, H), lambda i: (0, 0)),
            pl.BlockSpec((H, H), lambda i: (0, 0)),
            pl.BlockSpec((1, H), lambda i: (0, 0)),
            pl.BlockSpec((rows_per_block, P, H), lambda i: (i, 0, 0)),
        ],
        out_specs=pl.BlockSpec((rows_per_block, P, H), lambda i: (i, 0, 0)),
        out_shape=jax.ShapeDtypeStruct((B, P, H), jnp.float32),
        scratch_shapes=[pltpu.VMEM((100, H), jnp.float32)],
        compiler_params=pltpu.CompilerParams(
            dimension_semantics=("arbitrary",),
        ),
    )(dts2d, emb_table, w, b2d, x3)


def kernel(x, dts, emb_table, W, b):
    b0, b1, d2, d3, d4 = x.shape
    B = b0 * b1
    P = d2 * d3
    x3 = x.reshape(B, P, d4)
    dts2d = dts.reshape(B, 1)
    out = _run(x3, dts2d, emb_table, W, b.reshape(1, d4), rows_per_block=8)
    return out.reshape(b0, b1, d2, d3, d4)


# speedup vs baseline: 1.0625x; 1.0625x over previous
import functools

import jax
import jax.numpy as jnp
from jax import lax
from jax.experimental import pallas as pl
from jax.experimental.pallas import tpu as pltpu


def _temb_body(idx_ref, emb_ref, w_ref, b_ref, t_ref):
    table = lax.dot_general(
        emb_ref[...], w_ref[...],
        dimension_numbers=(((1,), (1,)), ((), ())),
        preferred_element_type=jnp.float32,
    ) + b_ref[...]
    ids = idx_ref[...] * 2
    oh = (ids == lax.broadcasted_iota(jnp.int32, (1, 100), 1)).astype(jnp.float32)
    t_ref[...] = lax.dot_general(
        oh, table,
        dimension_numbers=(((1,), (0,)), ((), ())),
        preferred_element_type=jnp.float32,
    )


def _add_body(t_ref, x_ref, o_ref):
    o_ref[...] = x_ref[...] + t_ref[...][:, None, :]


@functools.partial(jax.jit, static_argnames=("rows_per_block",))
def _run(x3, dts2d, emb_table, w, b2d, rows_per_block):
    B, P, H = x3.shape
    t_all = pl.pallas_call(
        _temb_body,
        out_shape=jax.ShapeDtypeStruct((B, H), jnp.float32),
    )(dts2d, emb_table, w, b2d)

    n_blocks = B // rows_per_block
    return pl.pallas_call(
        _add_body,
        grid=(n_blocks,),
        in_specs=[
            pl.BlockSpec((rows_per_block, H), lambda i: (i, 0)),
            pl.BlockSpec((rows_per_block, P, H), lambda i: (i, 0, 0)),
        ],
        out_specs=pl.BlockSpec((rows_per_block, P, H), lambda i: (i, 0, 0)),
        out_shape=jax.ShapeDtypeStruct((B, P, H), jnp.float32),
        compiler_params=pltpu.CompilerParams(
            dimension_semantics=("arbitrary",),
        ),
    )(t_all, x3)


def kernel(x, dts, emb_table, W, b):
    b0, b1, d2, d3, d4 = x.shape
    B = b0 * b1
    P = d2 * d3
    x3 = x.reshape(B, P, d4)
    dts2d = dts.reshape(B, 1)
    out = _run(x3, dts2d, emb_table, W, b.reshape(1, d4), rows_per_block=8)
    return out.reshape(b0, b1, d2, d3, d4)


# speedup vs baseline: 1.4910x; 1.4032x over previous
import functools

import jax
import jax.numpy as jnp
from jax import lax
from jax.experimental import pallas as pl
from jax.experimental.pallas import tpu as pltpu


def _temb_body(idx_ref, emb_ref, w_ref, b_ref, t_ref):
    table = lax.dot_general(
        emb_ref[...], w_ref[...],
        dimension_numbers=(((1,), (1,)), ((), ())),
        preferred_element_type=jnp.float32,
    ) + b_ref[...]
    ids = idx_ref[...] * 2
    oh = (ids == lax.broadcasted_iota(jnp.int32, (1, 100), 1)).astype(jnp.float32)
    t_ref[...] = lax.dot_general(
        oh, table,
        dimension_numbers=(((1,), (0,)), ((), ())),
        preferred_element_type=jnp.float32,
    )


def _add_body(t_ref, x_ref, o_ref):
    o_ref[...] = x_ref[...] + t_ref[...][:, None, :]


@functools.partial(jax.jit, static_argnames=("rows_per_block",))
def _run(x3, dts2d, emb_table, w, b2d, rows_per_block):
    B, P, H = x3.shape
    t_all = pl.pallas_call(
        _temb_body,
        out_shape=jax.ShapeDtypeStruct((B, H), jnp.float32),
    )(dts2d, emb_table, w, b2d)

    n_blocks = B // rows_per_block
    return pl.pallas_call(
        _add_body,
        grid=(n_blocks,),
        in_specs=[
            pl.BlockSpec((rows_per_block, H), lambda i: (i, 0)),
            pl.BlockSpec((rows_per_block, P, H), lambda i: (i, 0, 0)),
        ],
        out_specs=pl.BlockSpec((rows_per_block, P, H), lambda i: (i, 0, 0)),
        out_shape=jax.ShapeDtypeStruct((B, P, H), jnp.float32),
        compiler_params=pltpu.CompilerParams(
            dimension_semantics=("arbitrary",),
        ),
    )(t_all, x3)


def kernel(x, dts, emb_table, W, b):
    b0, b1, d2, d3, d4 = x.shape
    B = b0 * b1
    P = d2 * d3
    x3 = x.reshape(B, P, d4)
    dts2d = dts.reshape(B, 1)
    out = _run(x3, dts2d, emb_table, W, b.reshape(1, d4), rows_per_block=16)
    return out.reshape(b0, b1, d2, d3, d4)


# speedup vs baseline: 1.9817x; 1.3291x over previous
import functools

import jax
import jax.numpy as jnp
from jax import lax
from jax.experimental import pallas as pl
from jax.experimental.pallas import tpu as pltpu


def _temb_body(idx_ref, emb_ref, w_ref, b_ref, t_ref):
    table = lax.dot_general(
        emb_ref[...], w_ref[...],
        dimension_numbers=(((1,), (1,)), ((), ())),
        preferred_element_type=jnp.float32,
    ) + b_ref[...]
    ids = idx_ref[...] * 2
    oh = (ids == lax.broadcasted_iota(jnp.int32, (1, 100), 1)).astype(jnp.float32)
    t_ref[...] = lax.dot_general(
        oh, table,
        dimension_numbers=(((1,), (0,)), ((), ())),
        preferred_element_type=jnp.float32,
    )


def _add_body(t_ref, x_ref, o_ref):
    o_ref[...] = x_ref[...] + t_ref[...][:, None, :]


@functools.partial(jax.jit, static_argnames=("rows_per_block",))
def _run(x3, dts2d, emb_table, w, b2d, rows_per_block):
    B, P, H = x3.shape
    t_all = pl.pallas_call(
        _temb_body,
        out_shape=jax.ShapeDtypeStruct((B, H), jnp.float32),
    )(dts2d, emb_table, w, b2d)

    n_blocks = B // rows_per_block
    return pl.pallas_call(
        _add_body,
        grid=(n_blocks,),
        in_specs=[
            pl.BlockSpec((rows_per_block, H), lambda i: (i, 0)),
            pl.BlockSpec((rows_per_block, P, H), lambda i: (i, 0, 0)),
        ],
        out_specs=pl.BlockSpec((rows_per_block, P, H), lambda i: (i, 0, 0)),
        out_shape=jax.ShapeDtypeStruct((B, P, H), jnp.float32),
        compiler_params=pltpu.CompilerParams(
            dimension_semantics=("arbitrary",),
        ),
    )(t_all, x3)


def kernel(x, dts, emb_table, W, b):
    b0, b1, d2, d3, d4 = x.shape
    B = b0 * b1
    P = d2 * d3
    x3 = x.reshape(B, P, d4)
    dts2d = dts.reshape(B, 1)
    out = _run(x3, dts2d, emb_table, W, b.reshape(1, d4), rows_per_block=32)
    return out.reshape(b0, b1, d2, d3, d4)


# speedup vs baseline: 2.1674x; 1.0937x over previous
import functools

import jax
import jax.numpy as jnp
from jax import lax
from jax.experimental import pallas as pl
from jax.experimental.pallas import tpu as pltpu


def _temb_body(idx_ref, emb_ref, w_ref, b_ref, t_ref):
    table = lax.dot_general(
        emb_ref[...], w_ref[...],
        dimension_numbers=(((1,), (1,)), ((), ())),
        preferred_element_type=jnp.float32,
    ) + b_ref[...]
    ids = idx_ref[...] * 2
    oh = (ids == lax.broadcasted_iota(jnp.int32, (1, 100), 1)).astype(jnp.float32)
    t_ref[...] = lax.dot_general(
        oh, table,
        dimension_numbers=(((1,), (0,)), ((), ())),
        preferred_element_type=jnp.float32,
    )


def _add_body(t_ref, x_ref, o_ref):
    o_ref[...] = x_ref[...] + t_ref[...][:, None, :]


@functools.partial(jax.jit, static_argnames=("rows_per_block",))
def _run(x3, dts2d, emb_table, w, b2d, rows_per_block):
    B, P, H = x3.shape
    t_all = pl.pallas_call(
        _temb_body,
        out_shape=jax.ShapeDtypeStruct((B, H), jnp.float32),
    )(dts2d, emb_table, w, b2d)

    n_blocks = B // rows_per_block
    return pl.pallas_call(
        _add_body,
        grid=(n_blocks,),
        in_specs=[
            pl.BlockSpec((rows_per_block, H), lambda i: (i, 0)),
            pl.BlockSpec((rows_per_block, P, H), lambda i: (i, 0, 0)),
        ],
        out_specs=pl.BlockSpec((rows_per_block, P, H), lambda i: (i, 0, 0)),
        out_shape=jax.ShapeDtypeStruct((B, P, H), jnp.float32),
        compiler_params=pltpu.CompilerParams(
            dimension_semantics=("arbitrary",),
        ),
    )(t_all, x3)


def kernel(x, dts, emb_table, W, b):
    b0, b1, d2, d3, d4 = x.shape
    B = b0 * b1
    P = d2 * d3
    x3 = x.reshape(B, P, d4)
    dts2d = dts.reshape(B, 1)
    out = _run(x3, dts2d, emb_table, W, b.reshape(1, d4), rows_per_block=80)
    return out.reshape(b0, b1, d2, d3, d4)


# speedup vs baseline: 2.2311x; 1.0294x over previous
import functools

import jax
import jax.numpy as jnp
from jax import lax
from jax.experimental import pallas as pl
from jax.experimental.pallas import tpu as pltpu


def _temb_body(idx_ref, emb_ref, w_ref, b_ref, t_ref):
    table = lax.dot_general(
        emb_ref[...], w_ref[...],
        dimension_numbers=(((1,), (1,)), ((), ())),
        preferred_element_type=jnp.float32,
    ) + b_ref[...]
    ids = idx_ref[...] * 2
    oh = (ids == lax.broadcasted_iota(jnp.int32, (1, 100), 1)).astype(jnp.float32)
    t_ref[...] = lax.dot_general(
        oh, table,
        dimension_numbers=(((1,), (0,)), ((), ())),
        preferred_element_type=jnp.float32,
    )


def _add_body(t_ref, x_ref, o_ref):
    o_ref[...] = x_ref[...] + t_ref[...][:, None, :]


@functools.partial(jax.jit, static_argnames=("rows_per_block",))
def _run(x3, dts2d, emb_table, w, b2d, rows_per_block):
    B, P, H = x3.shape
    t_all = pl.pallas_call(
        _temb_body,
        out_shape=jax.ShapeDtypeStruct((B, H), jnp.float32),
    )(dts2d, emb_table, w, b2d)

    n_blocks = B // rows_per_block
    return pl.pallas_call(
        _add_body,
        grid=(n_blocks,),
        in_specs=[
            pl.BlockSpec((rows_per_block, H), lambda i: (i, 0)),
            pl.BlockSpec((rows_per_block, P, H), lambda i: (i, 0, 0)),
        ],
        out_specs=pl.BlockSpec((rows_per_block, P, H), lambda i: (i, 0, 0)),
        out_shape=jax.ShapeDtypeStruct((B, P, H), jnp.float32),
        compiler_params=pltpu.CompilerParams(
            dimension_semantics=("arbitrary",),
        ),
    )(t_all, x3)


def kernel(x, dts, emb_table, W, b):
    b0, b1, d2, d3, d4 = x.shape
    B = b0 * b1
    P = d2 * d3
    x3 = x.reshape(B, P, d4)
    dts2d = dts.reshape(B, 1)
    out = _run(x3, dts2d, emb_table, W, b.reshape(1, d4), rows_per_block=160)
    return out.reshape(b0, b1, d2, d3, d4)


# speedup vs baseline: 2.2525x; 1.0096x over previous
import functools

import jax
import jax.numpy as jnp
from jax import lax
from jax.experimental import pallas as pl
from jax.experimental.pallas import tpu as pltpu


def _temb_body(idx_ref, emb_ref, w_ref, b_ref, t_ref):
    table = lax.dot_general(
        emb_ref[...], w_ref[...],
        dimension_numbers=(((1,), (1,)), ((), ())),
        preferred_element_type=jnp.float32,
    ) + b_ref[...]
    ids = idx_ref[...] * 2
    oh = (ids == lax.broadcasted_iota(jnp.int32, (1, 100), 1)).astype(jnp.float32)
    t_ref[...] = lax.dot_general(
        oh, table,
        dimension_numbers=(((1,), (0,)), ((), ())),
        preferred_element_type=jnp.float32,
    )


def _add_body(t_ref, x_ref, o_ref):
    o_ref[...] = x_ref[...] + t_ref[...][:, None, :]


@functools.partial(jax.jit, static_argnames=("rows_per_block",))
def _run(x3, dts2d, emb_table, w, b2d, rows_per_block):
    B, P, H = x3.shape
    t_all = pl.pallas_call(
        _temb_body,
        out_shape=jax.ShapeDtypeStruct((B, H), jnp.float32),
    )(dts2d, emb_table, w, b2d)

    n_blocks = B // rows_per_block
    return pl.pallas_call(
        _add_body,
        grid=(n_blocks,),
        in_specs=[
            pl.BlockSpec((rows_per_block, H), lambda i: (i, 0)),
            pl.BlockSpec((rows_per_block, P, H), lambda i: (i, 0, 0)),
        ],
        out_specs=pl.BlockSpec((rows_per_block, P, H), lambda i: (i, 0, 0)),
        out_shape=jax.ShapeDtypeStruct((B, P, H), jnp.float32),
        compiler_params=pltpu.CompilerParams(
            dimension_semantics=("arbitrary",),
        ),
    )(t_all, x3)


def kernel(x, dts, emb_table, W, b):
    b0, b1, d2, d3, d4 = x.shape
    B = b0 * b1
    P = d2 * d3
    x3 = x.reshape(B, P, d4)
    dts2d = dts.reshape(B, 1)
    out = _run(x3, dts2d, emb_table, W, b.reshape(1, d4), rows_per_block=200)
    return out.reshape(b0, b1, d2, d3, d4)


# speedup vs baseline: 2.4661x; 1.0948x over previous
import functools

import jax
import jax.numpy as jnp
from jax import lax
from jax.experimental import pallas as pl
from jax.experimental.pallas import tpu as pltpu


def _make_body(rows_per_block):
    def _body(idx_ref, emb_ref, w_ref, b_ref, x_ref, o_ref, t_ref):
        i = pl.program_id(0)

        @pl.when(i == 0)
        def _():
            table = lax.dot_general(
                emb_ref[...], w_ref[...],
                dimension_numbers=(((1,), (1,)), ((), ())),
                preferred_element_type=jnp.float32,
            ) + b_ref[...]
            ids = idx_ref[...] * 2
            oh = (ids == lax.broadcasted_iota(jnp.int32, (1, 100), 1))
            t_ref[...] = lax.dot_general(
                oh.astype(jnp.float32), table,
                dimension_numbers=(((1,), (0,)), ((), ())),
                preferred_element_type=jnp.float32,
            )

        t_rows = t_ref[pl.ds(i * rows_per_block, rows_per_block), :]
        o_ref[...] = x_ref[...] + t_rows[:, None, :]

    return _body


@functools.partial(jax.jit, static_argnames=("rows_per_block",))
def _run(x3, dts2d, emb_table, w, b2d, rows_per_block):
    B, P, H = x3.shape
    n_blocks = B // rows_per_block
    return pl.pallas_call(
        _make_body(rows_per_block),
        grid=(n_blocks,),
        in_specs=[
            pl.BlockSpec((B, 1), lambda i: (0, 0)),
            pl.BlockSpec((100, H), lambda i: (0, 0)),
            pl.BlockSpec((H, H), lambda i: (0, 0)),
            pl.BlockSpec((1, H), lambda i: (0, 0)),
            pl.BlockSpec((rows_per_block, P, H), lambda i: (i, 0, 0)),
        ],
        out_specs=pl.BlockSpec((rows_per_block, P, H), lambda i: (i, 0, 0)),
        out_shape=jax.ShapeDtypeStruct((B, P, H), jnp.float32),
        scratch_shapes=[pltpu.VMEM((B, H), jnp.float32)],
        compiler_params=pltpu.CompilerParams(
            dimension_semantics=("arbitrary",),
        ),
    )(dts2d, emb_table, w, b2d, x3)


def kernel(x, dts, emb_table, W, b):
    b0, b1, d2, d3, d4 = x.shape
    B = b0 * b1
    P = d2 * d3
    x3 = x.reshape(B, P, d4)
    dts2d = dts.reshape(B, 1)
    out = _run(x3, dts2d, emb_table, W, b.reshape(1, d4), rows_per_block=200)
    return out.reshape(b0, b1, d2, d3, d4)
